# baseline (speedup 1.0000x reference)
import functools

import jax
import jax.numpy as jnp
from jax import lax
from jax.experimental import pallas as pl
from jax.experimental.pallas import tpu as pltpu
from jax.experimental.pallas import tpu_sc as plsc

N = 10000
E = 320000
D = 128
H = 64
T = 10

NC = 1
NS = 16
NW = NC * NS
CH = 128
NCH = 160
EPW = NCH * CH
E_PAD = NW * EPW
NBUF = 4
N_PAD = 10112
ZR = N_PAD // NS


def _mlp_a_body(x_ref, w1_ref, b1_ref, w2_ref, b2_ref, wh_ref, bh_ref,
                p_ref, h_ref):
    x = x_ref[...]
    t = jnp.tanh(jnp.dot(x, w1_ref[...], preferred_element_type=jnp.float32)
                 + b1_ref[...])
    filt = jax.nn.sigmoid(
        jnp.dot(t, w2_ref[...], preferred_element_type=jnp.float32)
        + b2_ref[...])
    h = jnp.maximum(
        jnp.dot(x, wh_ref[...], preferred_element_type=jnp.float32)
        + bh_ref[...], 0.0)
    h_ref[...] = h
    p_ref[...] = h * filt


_mlp_a = pl.pallas_call(
    _mlp_a_body,
    out_shape=(jax.ShapeDtypeStruct((N, H), jnp.float32),
               jax.ShapeDtypeStruct((N, H), jnp.float32)),
)


def _mlp_b_body(h_ref, agg_ref, wg_ref, bg_ref, wo_ref, bo_ref, out_ref):
    s = h_ref[...]
    for c in range(NC):
        s = s + agg_ref[c, :N]
    h2 = jnp.maximum(
        jnp.dot(s, wg_ref[...], preferred_element_type=jnp.float32)
        + bg_ref[...], 0.0)
    out_ref[...] = (jnp.dot(h2, wo_ref[...], preferred_element_type=jnp.float32)
                    + bo_ref[...])


_mlp_b = pl.pallas_call(
    _mlp_b_body,
    out_shape=jax.ShapeDtypeStruct((N, T), jnp.float32),
)


@functools.partial(
    pl.kernel,
    out_type=jax.ShapeDtypeStruct((NC, N_PAD, H), jnp.float32),
    mesh=plsc.VectorSubcoreMesh(core_axis_name="c", subcore_axis_name="s",
                                num_cores=NC),
    scratch_types=[
        pltpu.VMEM((NCH, CH), jnp.int32),
        pltpu.VMEM((NCH, CH), jnp.int32),
        pltpu.VMEM((NBUF, CH, H), jnp.float32),
        pltpu.VMEM_SHARED((N_PAD, H), jnp.float32),
        pltpu.SemaphoreType.DMA((NBUF,)),
        pltpu.SemaphoreType.DMA((NBUF,)),
    ],
    compiler_params=pltpu.CompilerParams(use_tc_tiling_on_sc=False),
)
def _sc_agg(p_hbm, src_hbm, dst_hbm, zeros_hbm, out_hbm,
            src_v, dst_v, rows_v, acc_shared, gsem, ssem):
    cid = lax.axis_index("c")
    sid = lax.axis_index("s")
    wid = cid * NS + sid
    pltpu.sync_copy(zeros_hbm.at[pl.ds(sid * ZR, ZR)],
                    acc_shared.at[pl.ds(sid * ZR, ZR)])
    pltpu.sync_copy(src_hbm.at[wid], src_v)
    pltpu.sync_copy(dst_hbm.at[wid], dst_v)
    plsc.subcore_barrier()

    for b in range(NBUF):
        pltpu.async_copy(p_hbm.at[src_v.at[b]], rows_v.at[b], gsem.at[b])

    @pl.loop(0, NCH, step=NBUF)
    def _(j):
        for b in range(NBUF):
            pltpu.make_async_copy(p_hbm.at[src_v.at[j + b]], rows_v.at[b],
                                  gsem.at[b]).wait()
            pltpu.async_copy(rows_v.at[b], acc_shared.at[dst_v.at[j + b]],
                             ssem.at[b], add=True)
            pltpu.make_async_copy(rows_v.at[b], acc_shared.at[dst_v.at[j + b]],
                                  ssem.at[b]).wait()

            @pl.when(j + b + NBUF < NCH)
            def _():
                pltpu.async_copy(p_hbm.at[src_v.at[j + b + NBUF]],
                                 rows_v.at[b], gsem.at[b])

    plsc.subcore_barrier()
    pltpu.sync_copy(acc_shared.at[pl.ds(sid * ZR, ZR)],
                    out_hbm.at[cid, pl.ds(sid * ZR, ZR)])


def kernel(x, edge_index, W1, b1, W2, b2, Wh, bh, Wg, bg, Wo, bo):
    p, h = _mlp_a(x, W1, b1.reshape(1, H), W2, b2.reshape(1, H),
                  Wh, bh.reshape(1, H))
    pad = E_PAD - E
    src = jnp.concatenate([edge_index[0], jnp.zeros((pad,), jnp.int32)])
    trash = N + (jnp.arange(pad, dtype=jnp.int32) % (N_PAD - N))
    dst = jnp.concatenate([edge_index[1], trash])
    zeros = jnp.zeros((N_PAD, H), jnp.float32)
    agg = _sc_agg(p, src.reshape(NW, NCH, CH), dst.reshape(NW, NCH, CH), zeros)
    return _mlp_b(h, agg, Wg, bg.reshape(1, H), Wo, bo.reshape(1, T))


# speedup vs baseline: 1.8101x; 1.8101x over previous
import functools

import jax
import jax.numpy as jnp
import numpy as np
from jax import lax
from jax.experimental import pallas as pl
from jax.experimental.pallas import tpu as pltpu
from jax.experimental.pallas import tpu_sc as plsc

N = 10000
E = 320000
D = 128
H = 64
T = 10

NC = 2
NS = 16
NW = NC * NS
CH = 128
NCH = 80
EPW = NCH * CH
E_PAD = NW * EPW
NBUF = 4
N_PAD = 10112
ZR = N_PAD // NS

_COLPOS = np.arange(H).reshape(2, 16, 2).transpose(0, 2, 1).reshape(H)
_PERM = np.argsort(_COLPOS)


def _mlp_a_body(x_ref, w1_ref, b1_ref, w2p_ref, b2p_ref, whp_ref, bhp_ref,
                wh_ref, bh_ref, p_ref, h_ref):
    x = x_ref[...]
    t = jnp.tanh(jnp.dot(x, w1_ref[...], preferred_element_type=jnp.float32)
                 + b1_ref[...])
    filt_p = jax.nn.sigmoid(
        jnp.dot(t, w2p_ref[...], preferred_element_type=jnp.float32)
        + b2p_ref[...])
    h_p = jnp.maximum(
        jnp.dot(x, whp_ref[...], preferred_element_type=jnp.float32)
        + bhp_ref[...], 0.0)
    h = jnp.maximum(
        jnp.dot(x, wh_ref[...], preferred_element_type=jnp.float32)
        + bh_ref[...], 0.0)
    h_ref[...] = h
    p_ref[:N] = (h_p * filt_p).astype(jnp.bfloat16)
    p_ref[N:] = jnp.zeros((N_PAD - N, H), jnp.bfloat16)


_mlp_a = pl.pallas_call(
    _mlp_a_body,
    out_shape=(jax.ShapeDtypeStruct((N_PAD, H), jnp.bfloat16),
               jax.ShapeDtypeStruct((N, H), jnp.float32)),
)


def _mlp_b_body(h_ref, agg_ref, wg_ref, bg_ref, wo_ref, bo_ref, out_ref):
    s = h_ref[...]
    for c in range(NC):
        s = s + agg_ref[c, :N]
    h2 = jnp.maximum(
        jnp.dot(s, wg_ref[...], preferred_element_type=jnp.float32)
        + bg_ref[...], 0.0)
    out_ref[...] = (jnp.dot(h2, wo_ref[...], preferred_element_type=jnp.float32)
                    + bo_ref[...])


_mlp_b = pl.pallas_call(
    _mlp_b_body,
    out_shape=jax.ShapeDtypeStruct((N, T), jnp.float32),
)


@functools.partial(
    pl.kernel,
    out_type=jax.ShapeDtypeStruct((NC, N_PAD, H), jnp.float32),
    mesh=plsc.VectorSubcoreMesh(core_axis_name="c", subcore_axis_name="s",
                                num_cores=NC),
    scratch_types=[
        pltpu.VMEM((NCH, CH), jnp.int32),
        pltpu.VMEM((NCH, CH), jnp.int32),
        pltpu.VMEM((NBUF, CH, H), jnp.bfloat16),
        pltpu.VMEM((NBUF, CH, H), jnp.float32),
        pltpu.VMEM_SHARED((N_PAD, H), jnp.float32),
        pltpu.SemaphoreType.DMA((NBUF,)),
        pltpu.SemaphoreType.DMA((NBUF,)),
    ],
    compiler_params=pltpu.CompilerParams(use_tc_tiling_on_sc=False,
                                         needs_layout_passes=False),
)
def _sc_agg(p_hbm, src_hbm, dst_hbm, zeros_hbm, out_hbm,
            src_v, dst_v, rows_bf, rows_f, acc_shared, gsem, ssem):
    cid = lax.axis_index("c")
    sid = lax.axis_index("s")
    wid = cid * NS + sid
    pltpu.sync_copy(zeros_hbm.at[pl.ds(sid * ZR, ZR)],
                    acc_shared.at[pl.ds(sid * ZR, ZR)])
    pltpu.sync_copy(src_hbm.at[wid], src_v)
    pltpu.sync_copy(dst_hbm.at[wid], dst_v)
    plsc.subcore_barrier()

    for b in range(NBUF):
        pltpu.async_copy(p_hbm.at[src_v.at[b]], rows_bf.at[b], gsem.at[b])

    @pl.loop(0, NCH, step=NBUF)
    def _(j):
        for b in range(NBUF):
            pltpu.make_async_copy(p_hbm.at[src_v.at[j + b]], rows_bf.at[b],
                                  gsem.at[b]).wait()

            @pl.when(j + b >= NBUF)
            def _():
                pltpu.make_async_copy(rows_f.at[b], acc_shared.at[dst_v.at[0]],
                                      ssem.at[b]).wait()

            @pl.loop(0, CH)
            def _(r):
                for g in range(2):
                    v = rows_bf.at[b][r, pl.ds(g * 32, 32)]
                    w = plsc.bitcast(v, jnp.int32)
                    lo = plsc.bitcast(w << jnp.int32(16), jnp.float32)
                    hi = plsc.bitcast(w & jnp.int32(-65536), jnp.float32)
                    rows_f.at[b][r, pl.ds(g * 32, 16)] = lo
                    rows_f.at[b][r, pl.ds(g * 32 + 16, 16)] = hi

            pltpu.async_copy(rows_f.at[b], acc_shared.at[dst_v.at[j + b]],
                             ssem.at[b], add=True)

            @pl.when(j + b + NBUF < NCH)
            def _():
                pltpu.async_copy(p_hbm.at[src_v.at[j + b + NBUF]],
                                 rows_bf.at[b], gsem.at[b])

    for b in range(NBUF):
        pltpu.make_async_copy(rows_f.at[b], acc_shared.at[dst_v.at[0]],
                              ssem.at[b]).wait()
    plsc.subcore_barrier()
    pltpu.sync_copy(acc_shared.at[pl.ds(sid * ZR, ZR)],
                    out_hbm.at[cid, pl.ds(sid * ZR, ZR)])


def kernel(x, edge_index, W1, b1, W2, b2, Wh, bh, Wg, bg, Wo, bo):
    p, h = _mlp_a(x, W1, b1.reshape(1, H), W2[:, _PERM],
                  b2[_PERM].reshape(1, H), Wh[:, _PERM],
                  bh[_PERM].reshape(1, H), Wh, bh.reshape(1, H))
    pad = E_PAD - E
    src = jnp.concatenate([edge_index[0], jnp.zeros((pad,), jnp.int32)])
    trash = N + (jnp.arange(pad, dtype=jnp.int32) % (N_PAD - N))
    dst = jnp.concatenate([edge_index[1], trash])
    zeros = jnp.zeros((N_PAD, H), jnp.float32)
    agg = _sc_agg(p, src.reshape(NW, NCH, CH), dst.reshape(NW, NCH, CH), zeros)
    return _mlp_b(h, agg, Wg, bg.reshape(1, H), Wo, bo.reshape(1, T))


# speedup vs baseline: 2.0494x; 1.1322x over previous
import functools

import jax
import jax.numpy as jnp
import numpy as np
from jax import lax
from jax.experimental import pallas as pl
from jax.experimental.pallas import tpu as pltpu
from jax.experimental.pallas import tpu_sc as plsc

N = 10000
E = 320000
D = 128
H = 64
T = 10

NC = 2
NS = 16
NW = NC * NS
CH = 80
NCH = 125
EPW = NCH * CH
NBUF = 5
N_PAD = 10112
ZR = N_PAD // NS

_COLPOS = np.arange(H).reshape(2, 16, 2).transpose(0, 2, 1).reshape(H)
_PERM = np.argsort(_COLPOS)


def _mlp_a_body(x_ref, w1_ref, b1_ref, w2p_ref, b2p_ref, whp_ref, bhp_ref,
                wh_ref, bh_ref, p_ref, h_ref):
    x = x_ref[...]
    t = jnp.tanh(jnp.dot(x, w1_ref[...], preferred_element_type=jnp.float32)
                 + b1_ref[...])
    filt_p = jax.nn.sigmoid(
        jnp.dot(t, w2p_ref[...], preferred_element_type=jnp.float32)
        + b2p_ref[...])
    h_p = jnp.maximum(
        jnp.dot(x, whp_ref[...], preferred_element_type=jnp.float32)
        + bhp_ref[...], 0.0)
    h = jnp.maximum(
        jnp.dot(x, wh_ref[...], preferred_element_type=jnp.float32)
        + bh_ref[...], 0.0)
    h_ref[...] = h
    p_ref[:N] = (h_p * filt_p).astype(jnp.bfloat16)
    p_ref[N:] = jnp.zeros((N_PAD - N, H), jnp.bfloat16)


_mlp_a = pl.pallas_call(
    _mlp_a_body,
    out_shape=(jax.ShapeDtypeStruct((N_PAD, H), jnp.bfloat16),
               jax.ShapeDtypeStruct((N, H), jnp.float32)),
)


def _mlp_b_body(h_ref, agg_ref, wg_ref, bg_ref, wo_ref, bo_ref, out_ref):
    s = h_ref[...]
    for c in range(NC):
        s = s + agg_ref[c, :N]
    h2 = jnp.maximum(
        jnp.dot(s, wg_ref[...], preferred_element_type=jnp.float32)
        + bg_ref[...], 0.0)
    out_ref[...] = (jnp.dot(h2, wo_ref[...], preferred_element_type=jnp.float32)
                    + bo_ref[...])


_mlp_b = pl.pallas_call(
    _mlp_b_body,
    out_shape=jax.ShapeDtypeStruct((N, T), jnp.float32),
)


@functools.partial(
    pl.kernel,
    out_type=jax.ShapeDtypeStruct((NC, N_PAD, H), jnp.float32),
    mesh=plsc.VectorSubcoreMesh(core_axis_name="c", subcore_axis_name="s",
                                num_cores=NC),
    scratch_types=[
        pltpu.VMEM((NCH, CH), jnp.int32),
        pltpu.VMEM((NCH, CH), jnp.int32),
        pltpu.VMEM((NBUF, CH, H), jnp.bfloat16),
        pltpu.VMEM((NBUF, CH, H), jnp.float32),
        pltpu.VMEM_SHARED((N_PAD, H), jnp.float32),
        pltpu.SemaphoreType.DMA((NBUF,)),
        pltpu.SemaphoreType.DMA((NBUF,)),
    ],
    compiler_params=pltpu.CompilerParams(use_tc_tiling_on_sc=False,
                                         needs_layout_passes=False),
)
def _sc_agg(p_hbm, edge_hbm, zeros_hbm, out_hbm,
            src_v, dst_v, rows_bf, rows_f, acc_shared, gsem, ssem):
    cid = lax.axis_index("c")
    sid = lax.axis_index("s")
    wid = cid * NS + sid
    pltpu.sync_copy(zeros_hbm.at[pl.ds(sid * ZR, ZR)],
                    acc_shared.at[pl.ds(sid * ZR, ZR)])
    pltpu.sync_copy(edge_hbm.at[0, wid], src_v)
    pltpu.sync_copy(edge_hbm.at[1, wid], dst_v)
    plsc.subcore_barrier()

    for b in range(NBUF):
        pltpu.async_copy(p_hbm.at[src_v.at[b]], rows_bf.at[b], gsem.at[b])

    @pl.loop(0, NCH, step=NBUF)
    def _(j):
        for b in range(NBUF):
            pltpu.make_async_copy(p_hbm.at[src_v.at[j + b]], rows_bf.at[b],
                                  gsem.at[b]).wait()

            @pl.when(j + b >= NBUF)
            def _():
                pltpu.make_async_copy(rows_f.at[b], acc_shared.at[dst_v.at[0]],
                                      ssem.at[b]).wait()

            @pl.loop(0, CH)
            def _(r):
                for g in range(2):
                    v = rows_bf.at[b][r, pl.ds(g * 32, 32)]
                    w = plsc.bitcast(v, jnp.int32)
                    lo = plsc.bitcast(w << jnp.int32(16), jnp.float32)
                    hi = plsc.bitcast(w & jnp.int32(-65536), jnp.float32)
                    rows_f.at[b][r, pl.ds(g * 32, 16)] = lo
                    rows_f.at[b][r, pl.ds(g * 32 + 16, 16)] = hi

            pltpu.async_copy(rows_f.at[b], acc_shared.at[dst_v.at[j + b]],
                             ssem.at[b], add=True)

            @pl.when(j + b + NBUF < NCH)
            def _():
                pltpu.async_copy(p_hbm.at[src_v.at[j + b + NBUF]],
                                 rows_bf.at[b], gsem.at[b])

    for b in range(NBUF):
        pltpu.make_async_copy(rows_f.at[b], acc_shared.at[dst_v.at[0]],
                              ssem.at[b]).wait()
    plsc.subcore_barrier()
    pltpu.sync_copy(acc_shared.at[pl.ds(sid * ZR, ZR)],
                    out_hbm.at[cid, pl.ds(sid * ZR, ZR)])


def kernel(x, edge_index, W1, b1, W2, b2, Wh, bh, Wg, bg, Wo, bo):
    p, h = _mlp_a(x, W1, b1.reshape(1, H), W2[:, _PERM],
                  b2[_PERM].reshape(1, H), Wh[:, _PERM],
                  bh[_PERM].reshape(1, H), Wh, bh.reshape(1, H))
    zeros = jnp.zeros((N_PAD, H), jnp.float32)
    agg = _sc_agg(p, edge_index.reshape(2, NW, NCH, CH), zeros)
    return _mlp_b(h, agg, Wg, bg.reshape(1, H), Wo, bo.reshape(1, T))


# speedup vs baseline: 2.0494x; 1.0000x over previous
import functools

import jax
import jax.numpy as jnp
import numpy as np
from jax import lax
from jax.experimental import pallas as pl
from jax.experimental.pallas import tpu as pltpu
from jax.experimental.pallas import tpu_sc as plsc

N = 10000
E = 320000
D = 128
H = 64
T = 10

NC = 2
NS = 16
NW = NC * NS
CH = 80
NCH = 125
EPW = NCH * CH
NBUF = 5
N_PAD = 10112
ZR = N_PAD // NS

_COLPOS = np.arange(H).reshape(2, 16, 2).transpose(0, 2, 1).reshape(H)
_PERM = np.argsort(_COLPOS)


def _mlp_a_body(x_ref, w1_ref, b1_ref, w2p_ref, b2p_ref, whp_ref, bhp_ref,
                wh_ref, bh_ref, p_ref, h_ref):
    x = x_ref[...]
    t = jnp.tanh(jnp.dot(x, w1_ref[...], preferred_element_type=jnp.float32)
                 + b1_ref[...])
    filt_p = jax.nn.sigmoid(
        jnp.dot(t, w2p_ref[...], preferred_element_type=jnp.float32)
        + b2p_ref[...])
    h_p = jnp.maximum(
        jnp.dot(x, whp_ref[...], preferred_element_type=jnp.float32)
        + bhp_ref[...], 0.0)
    h = jnp.maximum(
        jnp.dot(x, wh_ref[...], preferred_element_type=jnp.float32)
        + bh_ref[...], 0.0)
    h_ref[...] = h
    p_ref[:N] = (h_p * filt_p).astype(jnp.bfloat16)
    p_ref[N:] = jnp.zeros((N_PAD - N, H), jnp.bfloat16)


_mlp_a = pl.pallas_call(
    _mlp_a_body,
    out_shape=(jax.ShapeDtypeStruct((N_PAD, H), jnp.bfloat16),
               jax.ShapeDtypeStruct((N, H), jnp.float32)),
)


def _mlp_b_body(h_ref, agg_ref, wg_ref, bg_ref, wo_ref, bo_ref, out_ref):
    s = h_ref[...]
    for c in range(NC):
        s = s + agg_ref[c, :N]
    h2 = jnp.maximum(
        jnp.dot(s, wg_ref[...], preferred_element_type=jnp.float32)
        + bg_ref[...], 0.0)
    out_ref[...] = (jnp.dot(h2, wo_ref[...], preferred_element_type=jnp.float32)
                    + bo_ref[...])


_mlp_b = pl.pallas_call(
    _mlp_b_body,
    out_shape=jax.ShapeDtypeStruct((N, T), jnp.float32),
)


@functools.partial(
    pl.kernel,
    out_type=jax.ShapeDtypeStruct((NC, N_PAD, H), jnp.float32),
    mesh=plsc.VectorSubcoreMesh(core_axis_name="c", subcore_axis_name="s",
                                num_cores=NC),
    scratch_types=[
        pltpu.VMEM((EPW,), jnp.int32),
        pltpu.VMEM((EPW,), jnp.int32),
        pltpu.VMEM((NBUF, CH, H), jnp.bfloat16),
        pltpu.VMEM((NBUF, CH, H), jnp.float32),
        pltpu.VMEM_SHARED((N_PAD, H), jnp.float32),
        pltpu.SemaphoreType.DMA((NBUF,)),
        pltpu.SemaphoreType.DMA((NBUF,)),
    ],
    compiler_params=pltpu.CompilerParams(use_tc_tiling_on_sc=False,
                                         needs_layout_passes=False),
)
def _sc_agg(p_hbm, edge_hbm, zeros_hbm, out_hbm,
            src_v, dst_v, rows_bf, rows_f, acc_shared, gsem, ssem):
    cid = lax.axis_index("c")
    sid = lax.axis_index("s")
    wid = cid * NS + sid
    pltpu.sync_copy(zeros_hbm.at[pl.ds(sid * ZR, ZR)],
                    acc_shared.at[pl.ds(sid * ZR, ZR)])
    pltpu.sync_copy(edge_hbm.at[0, pl.ds(wid * EPW, EPW)], src_v)
    pltpu.sync_copy(edge_hbm.at[1, pl.ds(wid * EPW, EPW)], dst_v)
    plsc.subcore_barrier()

    for b in range(NBUF):
        pltpu.async_copy(p_hbm.at[src_v.at[pl.ds(b * CH, CH)]], rows_bf.at[b], gsem.at[b])

    @pl.loop(0, NCH, step=NBUF)
    def _(j):
        for b in range(NBUF):
            pltpu.make_async_copy(p_hbm.at[src_v.at[pl.ds((j + b) * CH, CH)]], rows_bf.at[b],
                                  gsem.at[b]).wait()

            @pl.when(j + b >= NBUF)
            def _():
                pltpu.make_async_copy(rows_f.at[b], acc_shared.at[dst_v.at[pl.ds(0, CH)]],
                                      ssem.at[b]).wait()

            @pl.loop(0, CH)
            def _(r):
                for g in range(2):
                    v = rows_bf.at[b][r, pl.ds(g * 32, 32)]
                    w = plsc.bitcast(v, jnp.int32)
                    lo = plsc.bitcast(w << jnp.int32(16), jnp.float32)
                    hi = plsc.bitcast(w & jnp.int32(-65536), jnp.float32)
                    rows_f.at[b][r, pl.ds(g * 32, 16)] = lo
                    rows_f.at[b][r, pl.ds(g * 32 + 16, 16)] = hi

            pltpu.async_copy(rows_f.at[b], acc_shared.at[dst_v.at[pl.ds((j + b) * CH, CH)]],
                             ssem.at[b], add=True)

            @pl.when(j + b + NBUF < NCH)
            def _():
                pltpu.async_copy(p_hbm.at[src_v.at[pl.ds((j + b + NBUF) * CH, CH)]],
                                 rows_bf.at[b], gsem.at[b])

    for b in range(NBUF):
        pltpu.make_async_copy(rows_f.at[b], acc_shared.at[dst_v.at[pl.ds(0, CH)]],
                              ssem.at[b]).wait()
    plsc.subcore_barrier()
    pltpu.sync_copy(acc_shared.at[pl.ds(sid * ZR, ZR)],
                    out_hbm.at[cid, pl.ds(sid * ZR, ZR)])


def kernel(x, edge_index, W1, b1, W2, b2, Wh, bh, Wg, bg, Wo, bo):
    p, h = _mlp_a(x, W1, b1.reshape(1, H), W2[:, _PERM],
                  b2[_PERM].reshape(1, H), Wh[:, _PERM],
                  bh[_PERM].reshape(1, H), Wh, bh.reshape(1, H))
    zeros = jnp.zeros((N_PAD, H), jnp.float32)
    agg = _sc_agg(p, edge_index, zeros)
    return _mlp_b(h, agg, Wg, bg.reshape(1, H), Wo, bo.reshape(1, T))


# speedup vs baseline: 2.2662x; 1.1058x over previous
import functools

import jax
import jax.numpy as jnp
import numpy as np
from jax import lax
from jax.experimental import pallas as pl
from jax.experimental.pallas import tpu as pltpu
from jax.experimental.pallas import tpu_sc as plsc

N = 10000
E = 320000
D = 128
H = 64
T = 10

NC = 2
NS = 16
NW = NC * NS
CH = 80
NCH = 125
EPW = NCH * CH
NBUF = 5
N_PAD = 10112
ZR = N_PAD // NS

_COLPOS = np.arange(H).reshape(2, 16, 2).transpose(0, 2, 1).reshape(H)
_PERM = np.argsort(_COLPOS)


def _mlp_a_body(x_ref, w1_ref, b1_ref, w2p_ref, b2p_ref, whp_ref, bhp_ref,
                wh_ref, bh_ref, p_ref, h_ref):
    x = x_ref[...]
    t = jnp.tanh(jnp.dot(x, w1_ref[...], preferred_element_type=jnp.float32)
                 + b1_ref[...])
    filt_p = jax.nn.sigmoid(
        jnp.dot(t, w2p_ref[...], preferred_element_type=jnp.float32)
        + b2p_ref[...])
    h_p = jnp.maximum(
        jnp.dot(x, whp_ref[...], preferred_element_type=jnp.float32)
        + bhp_ref[...], 0.0)
    h = jnp.maximum(
        jnp.dot(x, wh_ref[...], preferred_element_type=jnp.float32)
        + bh_ref[...], 0.0)
    h_ref[...] = h
    p_ref[:N] = (h_p * filt_p).astype(jnp.bfloat16)
    p_ref[N:] = jnp.zeros((N_PAD - N, H), jnp.bfloat16)


_mlp_a = pl.pallas_call(
    _mlp_a_body,
    out_shape=(jax.ShapeDtypeStruct((N_PAD, H), jnp.bfloat16),
               jax.ShapeDtypeStruct((N, H), jnp.float32)),
)


def _mlp_b_body(h_ref, agg_ref, wg_ref, bg_ref, wo_ref, bo_ref, out_ref):
    s = h_ref[...]
    for c in range(NC):
        s = s + agg_ref[:N, c * H:(c + 1) * H]
    h2 = jnp.maximum(
        jnp.dot(s, wg_ref[...], preferred_element_type=jnp.float32)
        + bg_ref[...], 0.0)
    out_ref[...] = (lax.dot_general(wo_ref[...], h2, (((0,), (1,)), ((), ())),
                    preferred_element_type=jnp.float32) + bo_ref[...])


_mlp_b = pl.pallas_call(
    _mlp_b_body,
    out_shape=jax.ShapeDtypeStruct((T, N), jnp.float32),
)


@functools.partial(
    pl.kernel,
    out_type=jax.ShapeDtypeStruct((N_PAD, NC * H), jnp.float32),
    mesh=plsc.VectorSubcoreMesh(core_axis_name="c", subcore_axis_name="s",
                                num_cores=NC),
    scratch_types=[
        pltpu.VMEM((EPW,), jnp.int32),
        pltpu.VMEM((EPW,), jnp.int32),
        pltpu.VMEM((NBUF, CH, H), jnp.bfloat16),
        pltpu.VMEM((NBUF, CH, H), jnp.float32),
        pltpu.VMEM_SHARED((N_PAD, H), jnp.float32),
        pltpu.SemaphoreType.DMA((NBUF,)),
        pltpu.SemaphoreType.DMA((NBUF,)),
    ],
    compiler_params=pltpu.CompilerParams(use_tc_tiling_on_sc=False,
                                         needs_layout_passes=False),
)
def _sc_agg(p_hbm, edge_hbm, zeros_hbm, out_hbm,
            src_v, dst_v, rows_bf, rows_f, acc_shared, gsem, ssem):
    cid = lax.axis_index("c")
    sid = lax.axis_index("s")
    wid = cid * NS + sid
    pltpu.sync_copy(zeros_hbm.at[pl.ds(sid * ZR, ZR)],
                    acc_shared.at[pl.ds(sid * ZR, ZR)])
    pltpu.sync_copy(edge_hbm.at[0, pl.ds(wid * EPW, EPW)], src_v)
    pltpu.sync_copy(edge_hbm.at[1, pl.ds(wid * EPW, EPW)], dst_v)
    plsc.subcore_barrier()

    for b in range(NBUF):
        pltpu.async_copy(p_hbm.at[src_v.at[pl.ds(b * CH, CH)]], rows_bf.at[b], gsem.at[b])

    @pl.loop(0, NCH, step=NBUF)
    def _(j):
        for b in range(NBUF):
            pltpu.make_async_copy(p_hbm.at[src_v.at[pl.ds((j + b) * CH, CH)]], rows_bf.at[b],
                                  gsem.at[b]).wait()

            @pl.when(j + b >= NBUF)
            def _():
                pltpu.make_async_copy(rows_f.at[b], acc_shared.at[dst_v.at[pl.ds(0, CH)]],
                                      ssem.at[b]).wait()

            @pl.loop(0, CH)
            def _(r):
                for g in range(2):
                    v = rows_bf.at[b][r, pl.ds(g * 32, 32)]
                    w = plsc.bitcast(v, jnp.int32)
                    lo = plsc.bitcast(w << jnp.int32(16), jnp.float32)
                    hi = plsc.bitcast(w & jnp.int32(-65536), jnp.float32)
                    rows_f.at[b][r, pl.ds(g * 32, 16)] = lo
                    rows_f.at[b][r, pl.ds(g * 32 + 16, 16)] = hi

            pltpu.async_copy(rows_f.at[b], acc_shared.at[dst_v.at[pl.ds((j + b) * CH, CH)]],
                             ssem.at[b], add=True)

            @pl.when(j + b + NBUF < NCH)
            def _():
                pltpu.async_copy(p_hbm.at[src_v.at[pl.ds((j + b + NBUF) * CH, CH)]],
                                 rows_bf.at[b], gsem.at[b])

    for b in range(NBUF):
        pltpu.make_async_copy(rows_f.at[b], acc_shared.at[dst_v.at[pl.ds(0, CH)]],
                              ssem.at[b]).wait()
    plsc.subcore_barrier()
    pltpu.sync_copy(acc_shared.at[pl.ds(sid * ZR, ZR)],
                    out_hbm.at[pl.ds(sid * ZR, ZR), pl.ds(cid * H, H)])


def kernel(x, edge_index, W1, b1, W2, b2, Wh, bh, Wg, bg, Wo, bo):
    p, h = _mlp_a(x, W1, b1.reshape(1, H), W2[:, _PERM],
                  b2[_PERM].reshape(1, H), Wh[:, _PERM],
                  bh[_PERM].reshape(1, H), Wh, bh.reshape(1, H))
    zeros = jnp.zeros((N_PAD, H), jnp.float32)
    agg = _sc_agg(p, edge_index, zeros)
    return _mlp_b(h, agg, Wg, bg.reshape(1, H), Wo, bo.reshape(T, 1)).T


# speedup vs baseline: 2.3452x; 1.0349x over previous
import functools

import jax
import jax.numpy as jnp
import numpy as np
from jax import lax
from jax.experimental import pallas as pl
from jax.experimental.pallas import tpu as pltpu
from jax.experimental.pallas import tpu_sc as plsc

N = 10000
E = 320000
D = 128
H = 64
T = 10

NC = 2
NS = 16
NW = NC * NS
CH = 80
NCH = 125
EPW = NCH * CH
NBUF = 5
N_PAD = 10112
ZR = N_PAD // NS

_COLPOS = np.arange(H).reshape(2, 16, 2).transpose(0, 2, 1).reshape(H)
_PERM = np.argsort(_COLPOS)


def _dot_rt(a, bt):
    return lax.dot_general(a, bt, (((1,), (1,)), ((), ())),
                           preferred_element_type=jnp.float32)


def _mlp_a_body(x_ref, w1t_ref, b1_ref, w2p_ref, b2p_ref, whpt_ref, bhp_ref,
                wht_ref, bh_ref, p_ref, h_ref):
    x = x_ref[...]
    t = jnp.tanh(_dot_rt(x, w1t_ref[...]) + b1_ref[...])
    filt_p = jax.nn.sigmoid(
        jnp.dot(t, w2p_ref[...], preferred_element_type=jnp.float32)
        + b2p_ref[...])
    h_p = jnp.maximum(_dot_rt(x, whpt_ref[...]) + bhp_ref[...], 0.0)
    h = jnp.maximum(_dot_rt(x, wht_ref[...]) + bh_ref[...], 0.0)
    h_ref[...] = h
    p_ref[:N] = (h_p * filt_p).astype(jnp.bfloat16)
    p_ref[N:] = jnp.zeros((N_PAD - N, H), jnp.bfloat16)


_mlp_a = pl.pallas_call(
    _mlp_a_body,
    out_shape=(jax.ShapeDtypeStruct((N_PAD, H), jnp.bfloat16),
               jax.ShapeDtypeStruct((N, H), jnp.float32)),
)


def _mlp_b_body(h_ref, agg_ref, wg_ref, bg_ref, wo_ref, bo_ref, out_ref):
    s = h_ref[...]
    for c in range(NC):
        s = s + agg_ref[:N, c * H:(c + 1) * H]
    h2 = jnp.maximum(
        jnp.dot(s, wg_ref[...], preferred_element_type=jnp.float32)
        + bg_ref[...], 0.0)
    out_ref[...] = (lax.dot_general(wo_ref[...], h2, (((0,), (1,)), ((), ())),
                    preferred_element_type=jnp.float32) + bo_ref[...])


_mlp_b = pl.pallas_call(
    _mlp_b_body,
    out_shape=jax.ShapeDtypeStruct((T, N), jnp.float32),
)


@functools.partial(
    pl.kernel,
    out_type=jax.ShapeDtypeStruct((N_PAD, NC * H), jnp.float32),
    mesh=plsc.VectorSubcoreMesh(core_axis_name="c", subcore_axis_name="s",
                                num_cores=NC),
    scratch_types=[
        pltpu.VMEM((EPW,), jnp.int32),
        pltpu.VMEM((EPW,), jnp.int32),
        pltpu.VMEM((NBUF, CH, H), jnp.bfloat16),
        pltpu.VMEM((NBUF, CH, H), jnp.float32),
        pltpu.VMEM_SHARED((N_PAD, H), jnp.float32),
        pltpu.SemaphoreType.DMA((NBUF,)),
        pltpu.SemaphoreType.DMA((NBUF,)),
    ],
    compiler_params=pltpu.CompilerParams(use_tc_tiling_on_sc=False,
                                         needs_layout_passes=False),
)
def _sc_agg(p_hbm, edge_hbm, zeros_hbm, out_hbm,
            src_v, dst_v, rows_bf, rows_f, acc_shared, gsem, ssem):
    cid = lax.axis_index("c")
    sid = lax.axis_index("s")
    wid = cid * NS + sid
    zc = pltpu.async_copy(zeros_hbm.at[pl.ds(sid * ZR, ZR)],
                          acc_shared.at[pl.ds(sid * ZR, ZR)], gsem.at[0])
    sc = pltpu.async_copy(edge_hbm.at[0, pl.ds(wid * EPW, EPW)], src_v,
                          gsem.at[1])
    dc = pltpu.async_copy(edge_hbm.at[1, pl.ds(wid * EPW, EPW)], dst_v,
                          gsem.at[2])
    zc.wait()
    sc.wait()
    dc.wait()
    plsc.subcore_barrier()

    for b in range(NBUF):
        pltpu.async_copy(p_hbm.at[src_v.at[pl.ds(b * CH, CH)]], rows_bf.at[b], gsem.at[b])

    @pl.loop(0, NCH, step=NBUF)
    def _(j):
        for b in range(NBUF):
            pltpu.make_async_copy(p_hbm.at[src_v.at[pl.ds((j + b) * CH, CH)]], rows_bf.at[b],
                                  gsem.at[b]).wait()

            @pl.when(j + b >= NBUF)
            def _():
                pltpu.make_async_copy(rows_f.at[b], acc_shared.at[dst_v.at[pl.ds(0, CH)]],
                                      ssem.at[b]).wait()

            @pl.loop(0, CH)
            def _(r):
                for g in range(2):
                    v = rows_bf.at[b][r, pl.ds(g * 32, 32)]
                    w = plsc.bitcast(v, jnp.int32)
                    lo = plsc.bitcast(w << jnp.int32(16), jnp.float32)
                    hi = plsc.bitcast(w & jnp.int32(-65536), jnp.float32)
                    rows_f.at[b][r, pl.ds(g * 32, 16)] = lo
                    rows_f.at[b][r, pl.ds(g * 32 + 16, 16)] = hi

            pltpu.async_copy(rows_f.at[b], acc_shared.at[dst_v.at[pl.ds((j + b) * CH, CH)]],
                             ssem.at[b], add=True)

            @pl.when(j + b + NBUF < NCH)
            def _():
                pltpu.async_copy(p_hbm.at[src_v.at[pl.ds((j + b + NBUF) * CH, CH)]],
                                 rows_bf.at[b], gsem.at[b])

    for b in range(NBUF):
        pltpu.make_async_copy(rows_f.at[b], acc_shared.at[dst_v.at[pl.ds(0, CH)]],
                              ssem.at[b]).wait()
    plsc.subcore_barrier()
    pltpu.sync_copy(acc_shared.at[pl.ds(sid * ZR, ZR)],
                    out_hbm.at[pl.ds(sid * ZR, ZR), pl.ds(cid * H, H)])


def kernel(x, edge_index, W1, b1, W2, b2, Wh, bh, Wg, bg, Wo, bo):
    p, h = _mlp_a(x, W1.T, b1.reshape(1, H), W2[:, _PERM],
                  b2[_PERM].reshape(1, H), Wh.T[_PERM],
                  bh[_PERM].reshape(1, H), Wh.T, bh.reshape(1, H))
    zeros = jnp.zeros((N_PAD, H), jnp.float32)
    agg = _sc_agg(p, edge_index, zeros)
    return _mlp_b(h, agg, Wg, bg.reshape(1, H), Wo, bo.reshape(T, 1)).T
